# Initial kernel scaffold; baseline (speedup 1.0000x reference)
import jax
import jax.numpy as jnp
from jax.experimental import pallas as pl


def kernel(adj_indices, adj_values, embeds):
    raise NotImplementedError("write your pallas kernel here")


# speedup vs baseline: 2.5681x; 2.5681x over previous
import functools

import jax
import jax.numpy as jnp
from jax import lax
from jax.experimental import pallas as pl
from jax.experimental.pallas import tpu as pltpu
from jax.experimental.pallas import tpu_sc as plsc

N = 10000
E = 320000
D = 128
SLOPE = 0.2

NC = 2
NS = 16
NW = NC * NS
C = 128
G = 79
E_PAD = NW * G * C
RPT = 632
N_PAD = NS * RPT


def _spmm_body(rows_hbm, cols_hbm, vals_hbm, embeds_hbm, out_hbm,
               idx_c, idx_r, vv, gbuf, acc, sem):
    cid = lax.axis_index("c")
    sid = lax.axis_index("s")
    wid = cid * NS + sid

    def _zrow(r, carry):
        for f in range(D // 16):
            gbuf[r, pl.ds(f * 16, 16)] = jnp.zeros((16,), jnp.float32)
        return carry
    lax.fori_loop(0, C, _zrow, 0)

    zbase = sid * RPT
    for k in range(RPT // C):
        pltpu.sync_copy(gbuf, acc.at[pl.ds(zbase + k * C, C)])
    rem = RPT % C
    pltpu.sync_copy(gbuf.at[pl.ds(0, rem)],
                    acc.at[pl.ds(zbase + (RPT // C) * C, rem)])
    plsc.subcore_barrier()

    def _chunk(g, carry):
        base = (wid * G + g) * C
        pltpu.sync_copy(cols_hbm.at[pl.ds(base, C)], idx_c)
        pltpu.sync_copy(rows_hbm.at[pl.ds(base, C)], idx_r)
        pltpu.sync_copy(vals_hbm.at[pl.ds(base * 16, C * 16)], vv)
        pltpu.async_copy(embeds_hbm.at[idx_c], gbuf, sem).wait()

        def _edge(e, c2):
            s = vv[pl.ds(e * 16, 16)]
            for f in range(D // 16):
                sl = pl.ds(f * 16, 16)
                gbuf[e, sl] = gbuf[e, sl] * s
            return c2
        lax.fori_loop(0, C, _edge, 0)

        pltpu.sync_copy(gbuf, acc.at[idx_r], add=True)
        return carry
    lax.fori_loop(0, G, _chunk, 0)

    plsc.subcore_barrier()

    obase = sid * RPT
    pltpu.sync_copy(acc.at[pl.ds(obase, RPT)],
                    out_hbm.at[cid, pl.ds(obase, RPT)])


_spmm_sc = functools.partial(
    pl.kernel,
    out_type=jax.ShapeDtypeStruct((NC, N_PAD, D), jnp.float32),
    mesh=plsc.VectorSubcoreMesh(core_axis_name="c", subcore_axis_name="s"),
    scratch_types=[
        pltpu.VMEM((C,), jnp.int32),
        pltpu.VMEM((C,), jnp.int32),
        pltpu.VMEM((C * 16,), jnp.float32),
        pltpu.VMEM((C, D), jnp.float32),
        pltpu.VMEM_SHARED((N_PAD, D), jnp.float32),
        pltpu.SemaphoreType.DMA,
    ],
)(_spmm_body)


def _combine_body(p_ref, o_ref):
    x = p_ref[0] + p_ref[1]
    o_ref[...] = jnp.where(x > 0, x, SLOPE * x)


def _combine(partials):
    bn = 632
    return pl.pallas_call(
        _combine_body,
        out_shape=jax.ShapeDtypeStruct((N_PAD, D), jnp.float32),
        grid=(N_PAD // bn,),
        in_specs=[pl.BlockSpec((NC, bn, D), lambda i: (0, i, 0))],
        out_specs=pl.BlockSpec((bn, D), lambda i: (i, 0)),
    )(partials)


def kernel(adj_indices, adj_values, embeds):
    rows = adj_indices[0].astype(jnp.int32)
    cols = adj_indices[1].astype(jnp.int32)
    vals = adj_values.astype(jnp.float32)
    pad = E_PAD - E
    rows = jnp.pad(rows, (0, pad))
    cols = jnp.pad(cols, (0, pad))
    vals = jnp.pad(vals, (0, pad))
    vals = jnp.broadcast_to(vals[:, None], (E_PAD, 16)).reshape(-1)
    partials = _spmm_sc(rows, cols, vals, embeds)
    return _combine(partials)[:N]
